# Initial kernel scaffold; baseline (speedup 1.0000x reference)
import jax
import jax.numpy as jnp
from jax.experimental import pallas as pl


def kernel(x, edge_index, Wl1, bl1, Wr1, Wl2, bl2, Wr2):
    raise NotImplementedError("write your pallas kernel here")


# speedup vs baseline: 1.0563x; 1.0563x over previous
import functools

import jax
import jax.numpy as jnp
from jax import lax
from jax.experimental import pallas as pl
from jax.experimental.pallas import tpu as pltpu
from jax.experimental.pallas import tpu_sc as plsc

N_NODES = 10000
N_EDGES = 320000
D = 128

NC = 2
NS = 16
NW = NC * NS
L = 16

NPW = 313
CHUNK = 2000
N_CHUNKS = N_EDGES // CHUNK
G = 16
TRASH = CHUNK + L


def _segmax_body(h_hbm, src_hbm, dst_hbm, out_hbm,
                 src_v, dst_v, msrc_v, mdst_v, rows_v, acc_v, sem):
    c = lax.axis_index("c")
    s = lax.axis_index("s")
    w = s * NC + c
    lo = w * NPW

    neg_inf = jnp.full((L,), -jnp.inf, jnp.float32)

    def init_row(r, _):
        for q in range(D // L):
            acc_v[r, pl.ds(q * L, L)] = neg_inf
        return 0
    lax.fori_loop(0, NPW + 1, init_row, 0)

    def chunk_body(ci, _):
        pltpu.sync_copy(src_hbm.at[pl.ds(ci * CHUNK, CHUNK)], src_v)
        pltpu.sync_copy(dst_hbm.at[pl.ds(ci * CHUNK, CHUNK)], dst_v)

        def scan_body(i, cnt):
            sv = src_v[pl.ds(i * L, L)]
            dv = dst_v[pl.ds(i * L, L)]
            m = (dv >= lo) & (dv < lo + NPW)
            pos = plsc.cumsum(m.astype(jnp.int32))
            offs = jnp.where(m, cnt + pos - 1, TRASH)
            plsc.store_scatter(msrc_v, [offs], sv)
            plsc.store_scatter(mdst_v, [offs], dv - lo)
            return cnt + pos[L - 1]
        cnt = lax.fori_loop(0, CHUNK // L, scan_body, 0)

        msrc_v[pl.ds(cnt, L)] = jnp.zeros((L,), jnp.int32)
        mdst_v[pl.ds(cnt, L)] = jnp.full((L,), NPW, jnp.int32)

        ngroups = (cnt + (G - 1)) // G

        def grp_body(g, _):
            idxv = msrc_v[pl.ds(g * G, G)]
            pltpu.async_copy(h_hbm.at[idxv], rows_v, sem).wait()
            dvec = mdst_v[pl.ds(g * G, G)]
            for e in range(G):
                d = dvec[e]
                for q in range(D // L):
                    rv = rows_v[e, pl.ds(q * L, L)]
                    av = acc_v[d, pl.ds(q * L, L)]
                    acc_v[d, pl.ds(q * L, L)] = jnp.maximum(av, rv)
            return 0
        lax.fori_loop(0, ngroups, grp_body, 0)
        return 0
    lax.fori_loop(0, N_CHUNKS, chunk_body, 0)

    def fin_row(r, _):
        for q in range(D // L):
            v = acc_v[r, pl.ds(q * L, L)]
            acc_v[r, pl.ds(q * L, L)] = jnp.where(v == -jnp.inf, 0.0, v)
        return 0
    lax.fori_loop(0, NPW, fin_row, 0)

    pltpu.sync_copy(acc_v.at[pl.ds(0, NPW)], out_hbm.at[w])


@functools.partial(
    pl.kernel,
    mesh=plsc.VectorSubcoreMesh(core_axis_name="c", subcore_axis_name="s"),
    compiler_params=pltpu.CompilerParams(needs_layout_passes=False),
    out_type=jax.ShapeDtypeStruct((NW, NPW, D), jnp.float32),
    scratch_types=[
        pltpu.VMEM((CHUNK,), jnp.int32),
        pltpu.VMEM((CHUNK,), jnp.int32),
        pltpu.VMEM((CHUNK + 2 * L,), jnp.int32),
        pltpu.VMEM((CHUNK + 2 * L,), jnp.int32),
        pltpu.VMEM((G, D), jnp.float32),
        pltpu.VMEM((NPW + 1, D), jnp.float32),
        pltpu.SemaphoreType.DMA,
    ],
)
def _segmax(h_hbm, src_hbm, dst_hbm, out_hbm,
            src_v, dst_v, msrc_v, mdst_v, rows_v, acc_v, sem):
    _segmax_body(h_hbm, src_hbm, dst_hbm, out_hbm,
                 src_v, dst_v, msrc_v, mdst_v, rows_v, acc_v, sem)


def _lin_body(agg_ref, h_ref, wl_ref, bl_ref, wr_ref, o_ref):
    a = lax.dot_general(agg_ref[...], wl_ref[...],
                        (((1,), (1,)), ((), ())),
                        preferred_element_type=jnp.float32)
    b = lax.dot_general(h_ref[...], wr_ref[...],
                        (((1,), (1,)), ((), ())),
                        preferred_element_type=jnp.float32)
    o_ref[...] = jnp.maximum(a + b + bl_ref[...], 0.0)


_ROWS_BLK = 400
_N_BLKS = N_NODES // _ROWS_BLK


def _linear(agg, h, Wl, bl, Wr):
    return pl.pallas_call(
        _lin_body,
        grid=(_N_BLKS,),
        in_specs=[
            pl.BlockSpec((_ROWS_BLK, D), lambda i: (i, 0)),
            pl.BlockSpec((_ROWS_BLK, D), lambda i: (i, 0)),
            pl.BlockSpec((D, D), lambda i: (0, 0)),
            pl.BlockSpec((1, D), lambda i: (0, 0)),
            pl.BlockSpec((D, D), lambda i: (0, 0)),
        ],
        out_specs=pl.BlockSpec((_ROWS_BLK, D), lambda i: (i, 0)),
        out_shape=jax.ShapeDtypeStruct((N_NODES, D), jnp.float32),
    )(agg, h, Wl, bl.reshape(1, D), Wr)


def kernel(x, edge_index, Wl1, bl1, Wr1, Wl2, bl2, Wr2):
    src = edge_index[0]
    dst = edge_index[1]
    agg1 = _segmax(x, src, dst).reshape(NW * NPW, D)[:N_NODES]
    h1 = _linear(agg1, x, Wl1, bl1, Wr1)
    agg2 = _segmax(h1, src, dst).reshape(NW * NPW, D)[:N_NODES]
    h2 = _linear(agg2, h1, Wl2, bl2, Wr2)
    return h2


# speedup vs baseline: 1.8094x; 1.7130x over previous
import functools

import jax
import jax.numpy as jnp
from jax import lax
from jax.experimental import pallas as pl
from jax.experimental.pallas import tpu as pltpu
from jax.experimental.pallas import tpu_sc as plsc

N_NODES = 10000
N_EDGES = 320000
D = 128

NC = 2
NS = 16
NW = NC * NS
L = 16

NPW = 313
CHUNK = 4000
N_CHUNKS = N_EDGES // CHUNK
G = 16
WIN = CHUNK + L
TRASH = CHUNK + L
LBLK = 4096
LROW = LBLK * 80
assert LROW >= N_EDGES + L * N_CHUNKS + CHUNK + L


def _issue_gather(h_hbm, src_ref, g, buf, sem):
    idxv = src_ref[pl.ds(g * G, G)]
    pltpu.make_async_copy(h_hbm.at[idxv], buf, sem).start()


def _wait_gather(h_hbm, src_ref, g, buf, sem):
    idxv = src_ref[pl.ds(g * G, G)]
    pltpu.make_async_copy(h_hbm.at[idxv], buf, sem).wait()


def _max_into_acc(dst_ref, g, buf, acc_v):
    dvec = dst_ref[pl.ds(g * G, G)]
    for e in range(G):
        d = dvec[e]
        for q in range(D // L):
            rv = buf[e, pl.ds(q * L, L)]
            av = acc_v[d, pl.ds(q * L, L)]
            acc_v[d, pl.ds(q * L, L)] = jnp.maximum(av, rv)


def _pipelined_groups(h_hbm, src_ref, dst_ref, ngroups,
                      rows2_v, sem0, sem1, acc_v):
    buf0 = rows2_v.at[0]
    buf1 = rows2_v.at[1]

    @pl.when(ngroups > 0)
    def _():
        _issue_gather(h_hbm, src_ref, 0, buf0, sem0)

    def pair(p, _):
        g0 = 2 * p
        g1 = g0 + 1
        _wait_gather(h_hbm, src_ref, g0, buf0, sem0)

        @pl.when(g1 < ngroups)
        def _():
            _issue_gather(h_hbm, src_ref, g1, buf1, sem1)
        _max_into_acc(dst_ref, g0, buf0, acc_v)

        @pl.when(g1 < ngroups)
        def _():
            _wait_gather(h_hbm, src_ref, g1, buf1, sem1)

            @pl.when(g1 + 1 < ngroups)
            def _():
                _issue_gather(h_hbm, src_ref, g1 + 1, buf0, sem0)
            _max_into_acc(dst_ref, g1, buf1, acc_v)
        return 0
    lax.fori_loop(0, (ngroups + 1) // 2, pair, 0)


def _init_acc(acc_v):
    neg_inf = jnp.full((L,), -jnp.inf, jnp.float32)

    def init_row(r, _):
        for q in range(D // L):
            acc_v[r, pl.ds(q * L, L)] = neg_inf
        return 0
    lax.fori_loop(0, NPW + 1, init_row, 0)


def _finalize_acc(acc_v, out_hbm, w):
    def fin_row(r, _):
        for q in range(D // L):
            v = acc_v[r, pl.ds(q * L, L)]
            acc_v[r, pl.ds(q * L, L)] = jnp.where(v == -jnp.inf, 0.0, v)
        return 0
    lax.fori_loop(0, NPW, fin_row, 0)
    pltpu.sync_copy(acc_v.at[pl.ds(0, NPW)], out_hbm.at[w])


@functools.partial(
    pl.kernel,
    mesh=plsc.VectorSubcoreMesh(core_axis_name="c", subcore_axis_name="s"),
    compiler_params=pltpu.CompilerParams(needs_layout_passes=False, use_tc_tiling_on_sc=False),
    out_type=(
        jax.ShapeDtypeStruct((NW, NPW, D), jnp.float32),
        jax.ShapeDtypeStruct((NW, LROW), jnp.int32),
        jax.ShapeDtypeStruct((NW, LROW), jnp.int32),
        jax.ShapeDtypeStruct((NW, L), jnp.int32),
    ),
    scratch_types=[
        pltpu.VMEM((CHUNK,), jnp.int32),
        pltpu.VMEM((CHUNK,), jnp.int32),
        pltpu.VMEM((CHUNK + 2 * L,), jnp.int32),
        pltpu.VMEM((CHUNK + 2 * L,), jnp.int32),
        pltpu.VMEM((2, G, D), jnp.float32),
        pltpu.VMEM((NPW + 1, D), jnp.float32),
        pltpu.VMEM((L,), jnp.int32),
        pltpu.SemaphoreType.DMA,
        pltpu.SemaphoreType.DMA,
        pltpu.SemaphoreType.DMA,
    ],
)
def _segmax_build(h_hbm, src_hbm, dst_hbm,
                  out_hbm, lsrc_hbm, ldst_hbm, counts_hbm,
                  src_v, dst_v, msrc_v, mdst_v, rows2_v, acc_v, cnt_v,
                  sem0, sem1, semw):
    c = lax.axis_index("c")
    s = lax.axis_index("s")
    w = s * NC + c
    lo = w * NPW

    _init_acc(acc_v)

    def chunk_body(ci, off):
        off = pl.multiple_of(off, L)
        pltpu.sync_copy(src_hbm.at[pl.ds(ci * CHUNK, CHUNK)], src_v)
        pltpu.sync_copy(dst_hbm.at[pl.ds(ci * CHUNK, CHUNK)], dst_v)

        def scan_body(i, cnt):
            sv = src_v[pl.ds(i * L, L)]
            dv = dst_v[pl.ds(i * L, L)]
            m = (dv >= lo) & (dv < lo + NPW)
            pos = plsc.cumsum(jnp.where(m, 1, 0))
            offs = jnp.where(m, cnt + pos - 1, TRASH)
            plsc.store_scatter(msrc_v, [offs], sv)
            plsc.store_scatter(mdst_v, [offs], dv - lo)
            return cnt + pos[L - 1]
        cnt = lax.fori_loop(0, CHUNK // L, scan_body, 0)

        msrc_v[pl.ds(cnt, L)] = jnp.zeros((L,), jnp.int32)
        mdst_v[pl.ds(cnt, L)] = jnp.full((L,), NPW, jnp.int32)

        ngroups = (cnt + (G - 1)) // G

        pltpu.make_async_copy(
            msrc_v.at[pl.ds(0, WIN)], lsrc_hbm.at[w, pl.ds(off, WIN)],
            semw).start()
        pltpu.make_async_copy(
            mdst_v.at[pl.ds(0, WIN)], ldst_hbm.at[w, pl.ds(off, WIN)],
            semw).start()

        _pipelined_groups(h_hbm, msrc_v, mdst_v, ngroups,
                          rows2_v, sem0, sem1, acc_v)

        pltpu.make_async_copy(
            msrc_v.at[pl.ds(0, WIN)], lsrc_hbm.at[w, pl.ds(off, WIN)],
            semw).wait()
        pltpu.make_async_copy(
            mdst_v.at[pl.ds(0, WIN)], ldst_hbm.at[w, pl.ds(off, WIN)],
            semw).wait()
        return off + ngroups * G
    total = lax.fori_loop(0, N_CHUNKS, chunk_body, 0)

    cnt_v[pl.ds(0, L)] = jnp.zeros((L,), jnp.int32) + total
    pltpu.sync_copy(cnt_v, counts_hbm.at[w])

    _finalize_acc(acc_v, out_hbm, w)


@functools.partial(
    pl.kernel,
    mesh=plsc.VectorSubcoreMesh(core_axis_name="c", subcore_axis_name="s"),
    compiler_params=pltpu.CompilerParams(needs_layout_passes=False, use_tc_tiling_on_sc=False),
    out_type=jax.ShapeDtypeStruct((NW, NPW, D), jnp.float32),
    scratch_types=[
        pltpu.VMEM((LBLK,), jnp.int32),
        pltpu.VMEM((LBLK,), jnp.int32),
        pltpu.VMEM((2, G, D), jnp.float32),
        pltpu.VMEM((NPW + 1, D), jnp.float32),
        pltpu.VMEM((L,), jnp.int32),
        pltpu.SemaphoreType.DMA,
        pltpu.SemaphoreType.DMA,
    ],
)
def _segmax_list(h_hbm, lsrc_hbm, ldst_hbm, counts_hbm, out_hbm,
                 lsrc_v, ldst_v, rows2_v, acc_v, cnt_v, sem0, sem1):
    c = lax.axis_index("c")
    s = lax.axis_index("s")
    w = s * NC + c

    _init_acc(acc_v)

    pltpu.sync_copy(counts_hbm.at[w], cnt_v)
    total = cnt_v[pl.ds(0, L)][0]

    nblocks = (total + LBLK - 1) // LBLK

    def block_body(b, _):
        boff = pl.multiple_of(b * LBLK, LBLK)
        pltpu.sync_copy(lsrc_hbm.at[w, pl.ds(boff, LBLK)], lsrc_v)
        pltpu.sync_copy(ldst_hbm.at[w, pl.ds(boff, LBLK)], ldst_v)
        nleft = total - b * LBLK
        ngroups = jnp.minimum(nleft, LBLK) // G
        _pipelined_groups(h_hbm, lsrc_v, ldst_v, ngroups,
                          rows2_v, sem0, sem1, acc_v)
        return 0
    lax.fori_loop(0, nblocks, block_body, 0)

    _finalize_acc(acc_v, out_hbm, w)


def _lin_body(agg_ref, h_ref, wl_ref, bl_ref, wr_ref, o_ref):
    a = lax.dot_general(agg_ref[...], wl_ref[...],
                        (((1,), (1,)), ((), ())),
                        preferred_element_type=jnp.float32)
    b = lax.dot_general(h_ref[...], wr_ref[...],
                        (((1,), (1,)), ((), ())),
                        preferred_element_type=jnp.float32)
    o_ref[...] = jnp.maximum(a + b + bl_ref[...], 0.0)


_ROWS_BLK = 400
_N_BLKS = N_NODES // _ROWS_BLK


def _linear(agg, h, Wl, bl, Wr):
    return pl.pallas_call(
        _lin_body,
        grid=(_N_BLKS,),
        in_specs=[
            pl.BlockSpec((_ROWS_BLK, D), lambda i: (i, 0)),
            pl.BlockSpec((_ROWS_BLK, D), lambda i: (i, 0)),
            pl.BlockSpec((D, D), lambda i: (0, 0)),
            pl.BlockSpec((1, D), lambda i: (0, 0)),
            pl.BlockSpec((D, D), lambda i: (0, 0)),
        ],
        out_specs=pl.BlockSpec((_ROWS_BLK, D), lambda i: (i, 0)),
        out_shape=jax.ShapeDtypeStruct((N_NODES, D), jnp.float32),
    )(agg, h, Wl, bl.reshape(1, D), Wr)


def kernel(x, edge_index, Wl1, bl1, Wr1, Wl2, bl2, Wr2):
    src = edge_index[0]
    dst = edge_index[1]
    agg1, lsrc, ldst, counts = _segmax_build(x, src, dst)
    h1 = _linear(agg1.reshape(NW * NPW, D)[:N_NODES], x, Wl1, bl1, Wr1)
    agg2 = _segmax_list(h1, lsrc, ldst, counts)
    h2 = _linear(agg2.reshape(NW * NPW, D)[:N_NODES], h1, Wl2, bl2, Wr2)
    return h2
